# baseline (speedup 1.0000x reference)
import functools

import jax
import jax.numpy as jnp
from jax import lax
from jax.experimental import pallas as pl
from jax.experimental.pallas import tpu as pltpu
from jax.experimental.pallas import tpu_sc as plsc

_E = 16
_NW = 32
_HS = 96
_TOK = 1024
_FL = _TOK * _E
_UNROLL = 2
_STEPS = _TOK // _E
_ROWS = 34
_ROW = _ROWS * _E
_T = 24576.0
_CAP = 2048.0
_LN2 = 0.6931471805599453
_LOGC = (3.342326876376589e-08, 1.0000030986470878, -0.5000129330593647,
         0.3330481239502715, -0.2491121064546097, 0.2061178523961052,
         -0.18627697325371723, 0.11448435452477138)

_CTOK = 256
_NCH = (3 * _TOK) // _CTOK
_CSTEPS = _CTOK // _E


def _sc_body(x_hbm, out_hbm, buf0, buf1, accs, sem0, sem1):
    cid = lax.axis_index("c")
    sid = lax.axis_index("s")
    wid = sid * 2 + cid
    iota = lax.iota(jnp.int32, _E)
    bufs = (buf0, buf1)
    sems = (sem0, sem1)
    cols = [jnp.full((_E,), j, jnp.int32) for j in range(_E)]

    def chunk_src(c):
        h = wid + _NW * (c >> 2)
        t0 = pl.multiple_of(h * _TOK + (c & 3) * _CTOK, _CTOK)
        return x_hbm.at[pl.ds(t0, _CTOK), :]

    def one_step(buf, rows):
        xs = [plsc.load_gather(buf, [rows, cols[j]]) for j in range(_E)]
        vals = xs
        idxs = [jnp.full((_E,), j, jnp.int32) for j in range(_E)]
        while len(vals) > 1:
            nv, ni = [], []
            for a in range(0, len(vals), 2):
                gt = vals[a + 1] > vals[a]
                nv.append(jnp.where(gt, vals[a + 1], vals[a]))
                ni.append(jnp.where(gt, idxs[a + 1], idxs[a]))
            vals, idxs = nv, ni
        bv, bi = vals[0], idxs[0]
        es = [jnp.exp(x - bv) for x in xs]
        ss = es
        while len(ss) > 1:
            ss = [ss[a] + ss[a + 1] for a in range(0, len(ss), 2)]
        s = ss[0]
        rs = 1.0 / s
        plsc.addupdate_scatter(
            accs, [bi + _E, iota], jnp.full((_E,), 1.0, jnp.float32))
        si = plsc.bitcast(s, jnp.int32)
        ee = (si - jnp.int32(0x3F3504F3)) >> 23
        mf = plsc.bitcast(si - (ee << 23), jnp.float32)
        u = mf - 1.0
        p = jnp.full((_E,), _LOGC[-1], jnp.float32)
        for c in _LOGC[-2::-1]:
            p = p * u + jnp.float32(c)
        logz = bv + (ee.astype(jnp.float32) * jnp.float32(_LN2) + p)
        return [e * rs for e in es], logz * logz

    def process_chunk(buf, acc):
        def step(i, carry):
            a2 = carry
            for uu in range(_UNROLL):
                rows = iota + (i * _UNROLL + uu) * _E
                dps, dz = one_step(buf, rows)
                a2 = [a + d for a, d in zip(a2, dps + [dz])]
            return a2
        return lax.fori_loop(0, _CSTEPS // _UNROLL, step, acc)

    pltpu.async_copy(chunk_src(jnp.int32(0)), buf0, sem0)

    def outer(c, carry):
        acc = carry
        h = wid + _NW * (c >> 2)
        pltpu.make_async_copy(chunk_src(c), buf0, sem0).wait()

        @pl.when((c & 3) == 0)
        def _zero():
            for r in range(_E, 2 * _E):
                accs[r, :] = jnp.zeros((_E,), jnp.float32)

        acc = process_chunk(buf0, acc)
        nc = c + 1

        @pl.when(nc < _NCH)
        def _next():
            pltpu.async_copy(chunk_src(nc), buf0, sem0)

        done = (c & 3) == 3

        @pl.when(done)
        def _flush():
            for j in range(_E):
                accs[j, :] = acc[j]
            accs[2 * _E, :] = acc[_E]
            pltpu.sync_copy(accs, out_hbm.at[h])

        return [jnp.where(done, 0.0, a) for a in acc]

    init = [jnp.zeros((_E,), jnp.float32)] * (_E + 1)
    lax.fori_loop(0, _NCH, outer, init)


_sc_stats = functools.partial(
    pl.kernel,
    mesh=plsc.VectorSubcoreMesh(core_axis_name="c", subcore_axis_name="s"),
    out_type=jax.ShapeDtypeStruct((_HS, _ROWS, _E), jnp.float32),
    scratch_types=[
        pltpu.VMEM((_CTOK, _E), jnp.float32),
        pltpu.VMEM((_CTOK, _E), jnp.float32),
        pltpu.VMEM((_ROWS, _E), jnp.float32),
        pltpu.SemaphoreType.DMA,
        pltpu.SemaphoreType.DMA,
    ],
    compiler_params=pltpu.CompilerParams(
        needs_layout_passes=False, skip_device_barrier=True),
)(_sc_body)


def _ep_body(d_ref, o_ref):
    d = d_ref[...]
    hh = lax.broadcasted_iota(jnp.int32, (_HS, 1, 1), 0)
    gh = (hh // 2) % 4
    ps = d[:, 0:_E, :]
    cs = d[:, _E:2 * _E, :]
    zsum = jnp.sum(d[:, 2 * _E:2 * _E + 1, :])
    pg, cg = [], []
    for g in range(4):
        mk = (gh == g).astype(jnp.float32)
        pg.append(jnp.sum(jnp.sum(ps * mk, axis=0), axis=1))
        cg.append(jnp.sum(jnp.sum(cs * mk, axis=0), axis=1))
    prob = jnp.stack(pg)
    cnt = jnp.stack(cg)
    over = jnp.maximum(cnt - _CAP, 0.0)
    keep = jnp.minimum(cnt, _CAP)
    extra = jnp.sum(over, axis=1, keepdims=True)
    e0 = (lax.broadcasted_iota(jnp.int32, (4, _E), 1) == 0).astype(jnp.float32)
    cadj = keep + extra * e0
    aux = jnp.sum(cadj * prob) * (_E * _E / (4.0 * _E * _T * _T))
    z = zsum / (4.0 * _T)
    o_ref[...] = jnp.full((1, 1), 0.001 * (z + aux), jnp.float32)


def kernel(router_outputs, attention_mask):
    del attention_mask
    parts = _sc_stats(router_outputs.reshape(_HS * _TOK, _E))
    out = pl.pallas_call(
        _ep_body,
        out_shape=jax.ShapeDtypeStruct((1, 1), jnp.float32),
    )(parts)
    return out[0, 0]


# speedup vs baseline: 1.0434x; 1.0434x over previous
import functools

import jax
import jax.numpy as jnp
from jax import lax
from jax.experimental import pallas as pl
from jax.experimental.pallas import tpu as pltpu
from jax.experimental.pallas import tpu_sc as plsc

_E = 16
_NW = 32
_HS = 96
_TOK = 1024
_ROWL = 128
_HROWS = _TOK * _E // _ROWL
_UNROLL = 2
_STEPS = _TOK // _E
_ROWS = 34
_T = 24576.0
_CAP = 2048.0
_LN2 = 0.6931471805599453
_LOGC = (3.342326876376589e-08, 1.0000030986470878, -0.5000129330593647,
         0.3330481239502715, -0.2491121064546097, 0.2061178523961052,
         -0.18627697325371723, 0.11448435452477138)


def _sc_body(x_hbm, out_hbm, buf0, buf1, accs, sem0, sem1):
    cid = lax.axis_index("c")
    sid = lax.axis_index("s")
    wid = sid * 2 + cid
    iota = lax.iota(jnp.int32, _E)
    riota = iota >> 3
    coff = (iota & 7) * _E
    bufs = (buf0, buf1)
    sems = (sem0, sem1)

    def hs_src(h):
        r0 = pl.multiple_of(h * _HROWS, _HROWS)
        return x_hbm.at[pl.ds(r0, _HROWS), :]

    def one_step(buf, st):
        rows = riota + 2 * st
        cls = [coff + j for j in range(_E)]
        xs = [plsc.load_gather(buf, [rows, cls[j]]) for j in range(_E)]
        vals = xs
        idxs = [jnp.full((_E,), j, jnp.int32) for j in range(_E)]
        while len(vals) > 1:
            nv, ni = [], []
            for a in range(0, len(vals), 2):
                gt = vals[a + 1] > vals[a]
                nv.append(jnp.where(gt, vals[a + 1], vals[a]))
                ni.append(jnp.where(gt, idxs[a + 1], idxs[a]))
            vals, idxs = nv, ni
        bv, bi = vals[0], idxs[0]
        es = [jnp.exp(x - bv) for x in xs]
        ss = es
        while len(ss) > 1:
            ss = [ss[a] + ss[a + 1] for a in range(0, len(ss), 2)]
        s = ss[0]
        rs = 1.0 / s
        plsc.addupdate_scatter(
            accs, [bi + _E, iota], jnp.full((_E,), 1.0, jnp.float32))
        si = plsc.bitcast(s, jnp.int32)
        ee = (si - jnp.int32(0x3F3504F3)) >> 23
        mf = plsc.bitcast(si - (ee << 23), jnp.float32)
        u = mf - 1.0
        p = jnp.full((_E,), _LOGC[-1], jnp.float32)
        for c in _LOGC[-2::-1]:
            p = p * u + jnp.float32(c)
        logz = bv + (ee.astype(jnp.float32) * jnp.float32(_LN2) + p)
        return [e * rs for e in es], logz * logz

    nk = _HS // _NW
    cps = {0: pltpu.async_copy(hs_src(wid), buf0, sem0)}
    for k in range(nk):
        h = wid + _NW * k
        if k + 1 < nk:
            cps[k + 1] = pltpu.async_copy(
                hs_src(wid + _NW * (k + 1)), bufs[(k + 1) % 2],
                sems[(k + 1) % 2])
        cps.pop(k).wait()
        buf = bufs[k % 2]
        for r in range(_E, 2 * _E):
            accs[r, :] = jnp.zeros((_E,), jnp.float32)

        def step(i, carry):
            a2 = carry
            for uu in range(_UNROLL):
                dps, dz = one_step(buf, i * _UNROLL + uu)
                a2 = [a + d for a, d in zip(a2, dps + [dz])]
            return a2

        init = [jnp.zeros((_E,), jnp.float32)] * (_E + 1)
        acc = lax.fori_loop(0, _STEPS // _UNROLL, step, init)
        for j in range(_E):
            accs[j, :] = acc[j]
        accs[2 * _E, :] = acc[_E]
        pltpu.sync_copy(accs, out_hbm.at[h])


_sc_stats = functools.partial(
    pl.kernel,
    mesh=plsc.VectorSubcoreMesh(core_axis_name="c", subcore_axis_name="s"),
    out_type=jax.ShapeDtypeStruct((_HS, _ROWS, _E), jnp.float32),
    scratch_types=[
        pltpu.VMEM((_HROWS, _ROWL), jnp.float32),
        pltpu.VMEM((_HROWS, _ROWL), jnp.float32),
        pltpu.VMEM((_ROWS, _E), jnp.float32),
        pltpu.SemaphoreType.DMA,
        pltpu.SemaphoreType.DMA,
    ],
    compiler_params=pltpu.CompilerParams(
        needs_layout_passes=False, skip_device_barrier=True),
)(_sc_body)


def _ep_body(d_ref, o_ref):
    d = d_ref[...]
    hh = lax.broadcasted_iota(jnp.int32, (_HS, 1, 1), 0)
    gh = (hh // 2) % 4
    ps = d[:, 0:_E, :]
    cs = d[:, _E:2 * _E, :]
    zsum = jnp.sum(d[:, 2 * _E:2 * _E + 1, :])
    pg, cg = [], []
    for g in range(4):
        mk = (gh == g).astype(jnp.float32)
        pg.append(jnp.sum(jnp.sum(ps * mk, axis=0), axis=1))
        cg.append(jnp.sum(jnp.sum(cs * mk, axis=0), axis=1))
    prob = jnp.stack(pg)
    cnt = jnp.stack(cg)
    over = jnp.maximum(cnt - _CAP, 0.0)
    keep = jnp.minimum(cnt, _CAP)
    extra = jnp.sum(over, axis=1, keepdims=True)
    e0 = (lax.broadcasted_iota(jnp.int32, (4, _E), 1) == 0).astype(jnp.float32)
    cadj = keep + extra * e0
    aux = jnp.sum(cadj * prob) * (_E * _E / (4.0 * _E * _T * _T))
    z = zsum / (4.0 * _T)
    o_ref[...] = jnp.full((1, 1), 0.001 * (z + aux), jnp.float32)


def kernel(router_outputs, attention_mask):
    del attention_mask
    parts = _sc_stats(router_outputs.reshape(_HS * _HROWS, _ROWL))
    out = pl.pallas_call(
        _ep_body,
        out_shape=jax.ShapeDtypeStruct((1, 1), jnp.float32),
    )(parts)
    return out[0, 0]


# speedup vs baseline: 1.0590x; 1.0150x over previous
import functools

import jax
import jax.numpy as jnp
from jax import lax
from jax.experimental import pallas as pl
from jax.experimental.pallas import tpu as pltpu
from jax.experimental.pallas import tpu_sc as plsc

_E = 16
_NW = 32
_HS = 96
_TOK = 1024
_ROWL = 128
_HROWS = _TOK * _E // _ROWL
_UNROLL = 2
_STEPS = _TOK // _E
_ROWS = 34
_T = 24576.0
_CAP = 2048.0
_LN2 = 0.6931471805599453
_LOGC = (3.342326876376589e-08, 1.0000030986470878, -0.5000129330593647,
         0.3330481239502715, -0.2491121064546097, 0.2061178523961052,
         -0.18627697325371723, 0.11448435452477138)

_CTOK = 256
_CROWS = _CTOK * _E // _ROWL
_NCH = (3 * _TOK) // _CTOK
_CSTEPS = _CTOK // _E


def _sc_body(x_hbm, out_hbm, buf0, buf1, accs, sem0, sem1):
    cid = lax.axis_index("c")
    sid = lax.axis_index("s")
    wid = sid * 2 + cid
    iota = lax.iota(jnp.int32, _E)
    riota = iota >> 3
    coff = (iota & 7) * _E
    bufs = (buf0, buf1)
    sems = (sem0, sem1)

    def chunk_src(c):
        h = wid + _NW * (c >> 2)
        r0 = pl.multiple_of(h * _HROWS + (c & 3) * _CROWS, _CROWS)
        return x_hbm.at[pl.ds(r0, _CROWS), :]

    def one_step(buf, st):
        rows = riota + 2 * st
        cls = [coff + j for j in range(_E)]
        xs = [plsc.load_gather(buf, [rows, cls[j]]) for j in range(_E)]
        vals = xs
        idxs = [jnp.full((_E,), j, jnp.int32) for j in range(_E)]
        while len(vals) > 1:
            nv, ni = [], []
            for a in range(0, len(vals), 2):
                gt = vals[a + 1] > vals[a]
                nv.append(jnp.where(gt, vals[a + 1], vals[a]))
                ni.append(jnp.where(gt, idxs[a + 1], idxs[a]))
            vals, idxs = nv, ni
        bv, bi = vals[0], idxs[0]
        es = [jnp.exp(x - bv) for x in xs]
        ss = es
        while len(ss) > 1:
            ss = [ss[a] + ss[a + 1] for a in range(0, len(ss), 2)]
        s = ss[0]
        rs = 1.0 / s
        plsc.addupdate_scatter(
            accs, [bi + _E, iota], jnp.full((_E,), 1.0, jnp.float32))
        si = plsc.bitcast(s, jnp.int32)
        ee = (si - jnp.int32(0x3F3504F3)) >> 23
        mf = plsc.bitcast(si - (ee << 23), jnp.float32)
        u = mf - 1.0
        p = jnp.full((_E,), _LOGC[-1], jnp.float32)
        for c in _LOGC[-2::-1]:
            p = p * u + jnp.float32(c)
        logz = bv + (ee.astype(jnp.float32) * jnp.float32(_LN2) + p)
        return [e * rs for e in es], logz * logz

    def process_chunk(buf, acc):
        def step(i, carry):
            a2 = carry
            for uu in range(_UNROLL):
                dps, dz = one_step(buf, i * _UNROLL + uu)
                a2 = [a + d for a, d in zip(a2, dps + [dz])]
            return a2
        return lax.fori_loop(0, _CSTEPS // _UNROLL, step, acc)

    pltpu.async_copy(chunk_src(jnp.int32(0)), buf0, sem0)
    pltpu.async_copy(chunk_src(jnp.int32(1)), buf1, sem1)

    def outer(ci, carry):
        acc = carry
        for b in range(2):
            c = 2 * ci + b
            h = wid + _NW * (c >> 2)
            pltpu.make_async_copy(chunk_src(c), bufs[b], sems[b]).wait()

            @pl.when((c & 3) == 0)
            def _zero():
                for r in range(_E, 2 * _E):
                    accs[r, :] = jnp.zeros((_E,), jnp.float32)

            acc = process_chunk(bufs[b], acc)
            nc = c + 2

            @pl.when(nc < _NCH)
            def _next():
                pltpu.async_copy(chunk_src(nc), bufs[b], sems[b])

            done = (c & 3) == 3

            @pl.when(done)
            def _flush():
                for j in range(_E):
                    accs[j, :] = acc[j]
                accs[2 * _E, :] = acc[_E]
                pltpu.sync_copy(accs, out_hbm.at[h])

            acc = [jnp.where(done, 0.0, a) for a in acc]
        return acc

    init = [jnp.zeros((_E,), jnp.float32)] * (_E + 1)
    lax.fori_loop(0, _NCH // 2, outer, init)


_sc_stats = functools.partial(
    pl.kernel,
    mesh=plsc.VectorSubcoreMesh(core_axis_name="c", subcore_axis_name="s"),
    out_type=jax.ShapeDtypeStruct((_HS, _ROWS, _E), jnp.float32),
    scratch_types=[
        pltpu.VMEM((_CROWS, _ROWL), jnp.float32),
        pltpu.VMEM((_CROWS, _ROWL), jnp.float32),
        pltpu.VMEM((_ROWS, _E), jnp.float32),
        pltpu.SemaphoreType.DMA,
        pltpu.SemaphoreType.DMA,
    ],
    compiler_params=pltpu.CompilerParams(
        needs_layout_passes=False, skip_device_barrier=True),
)(_sc_body)


def _ep_body(d_ref, o_ref):
    d = d_ref[...]
    hh = lax.broadcasted_iota(jnp.int32, (_HS, 1, 1), 0)
    gh = (hh // 2) % 4
    ps = d[:, 0:_E, :]
    cs = d[:, _E:2 * _E, :]
    zsum = jnp.sum(d[:, 2 * _E:2 * _E + 1, :])
    pg, cg = [], []
    for g in range(4):
        mk = (gh == g).astype(jnp.float32)
        pg.append(jnp.sum(jnp.sum(ps * mk, axis=0), axis=1))
        cg.append(jnp.sum(jnp.sum(cs * mk, axis=0), axis=1))
    prob = jnp.stack(pg)
    cnt = jnp.stack(cg)
    over = jnp.maximum(cnt - _CAP, 0.0)
    keep = jnp.minimum(cnt, _CAP)
    extra = jnp.sum(over, axis=1, keepdims=True)
    e0 = (lax.broadcasted_iota(jnp.int32, (4, _E), 1) == 0).astype(jnp.float32)
    cadj = keep + extra * e0
    aux = jnp.sum(cadj * prob) * (_E * _E / (4.0 * _E * _T * _T))
    z = zsum / (4.0 * _T)
    o_ref[...] = jnp.full((1, 1), 0.001 * (z + aux), jnp.float32)


def kernel(router_outputs, attention_mask):
    del attention_mask
    parts = _sc_stats(router_outputs.reshape(_HS * _HROWS, _ROWL))
    out = pl.pallas_call(
        _ep_body,
        out_shape=jax.ShapeDtypeStruct((1, 1), jnp.float32),
    )(parts)
    return out[0, 0]


# speedup vs baseline: 1.2926x; 1.2206x over previous
import functools

import jax
import jax.numpy as jnp
from jax import lax
from jax.experimental import pallas as pl
from jax.experimental.pallas import tpu as pltpu
from jax.experimental.pallas import tpu_sc as plsc

_E = 16
_NW = 32
_HS = 96
_TOK = 1024
_UNROLL = 2
_ROWS = 34
_T = 24576.0
_CAP = 2048.0
_LN2 = 0.6931471805599453
_LOGC = (3.342326876376589e-08, 1.0000030986470878, -0.5000129330593647,
         0.3330481239502715, -0.2491121064546097, 0.2061178523961052,
         -0.18627697325371723, 0.11448435452477138)

_CTOK = 256
_NCH = (3 * _TOK) // _CTOK
_CSTEPS = _CTOK // _E


def _sc_body(x_hbm, out_hbm, buf0, buf1, accs, sem0, sem1):
    cid = lax.axis_index("c")
    sid = lax.axis_index("s")
    wid = sid * 2 + cid
    iota = lax.iota(jnp.int32, _E)
    bufs = (buf0, buf1)
    sems = (sem0, sem1)
    cols = [jnp.full((_E,), j, jnp.int32) for j in range(_E)]

    def chunk_src(c):
        h = wid + _NW * (c >> 2)
        t0 = pl.multiple_of(h * _TOK + (c & 3) * _CTOK, _CTOK)
        return x_hbm.at[pl.ds(t0, _CTOK), :]

    def one_step(buf, rows):
        xs = [plsc.load_gather(buf, [rows, cols[j]]) for j in range(_E)]
        vals = xs
        idxs = [jnp.full((_E,), j, jnp.int32) for j in range(_E)]
        while len(vals) > 1:
            nv, ni = [], []
            for a in range(0, len(vals), 2):
                gt = vals[a + 1] > vals[a]
                nv.append(jnp.where(gt, vals[a + 1], vals[a]))
                ni.append(jnp.where(gt, idxs[a + 1], idxs[a]))
            vals, idxs = nv, ni
        bv, bi = vals[0], idxs[0]
        es = [jnp.exp(x - bv) for x in xs]
        ss = es
        while len(ss) > 1:
            ss = [ss[a] + ss[a + 1] for a in range(0, len(ss), 2)]
        s = ss[0]
        rs = 1.0 / s
        plsc.addupdate_scatter(
            accs, [bi + _E, iota], jnp.full((_E,), 1.0, jnp.float32))
        si = plsc.bitcast(s, jnp.int32)
        ee = (si - jnp.int32(0x3F3504F3)) >> 23
        mf = plsc.bitcast(si - (ee << 23), jnp.float32)
        u = mf - 1.0
        p = jnp.full((_E,), _LOGC[-1], jnp.float32)
        for c in _LOGC[-2::-1]:
            p = p * u + jnp.float32(c)
        logz = bv + (ee.astype(jnp.float32) * jnp.float32(_LN2) + p)
        return [e * rs for e in es], logz * logz

    def process_chunk(buf, acc):
        def step(i, carry):
            a2 = carry
            for uu in range(_UNROLL):
                rows = iota + (i * _UNROLL + uu) * _E
                dps, dz = one_step(buf, rows)
                a2 = [a + d for a, d in zip(a2, dps + [dz])]
            return a2
        return lax.fori_loop(0, _CSTEPS // _UNROLL, step, acc)

    pltpu.async_copy(chunk_src(jnp.int32(0)), buf0, sem0)
    pltpu.async_copy(chunk_src(jnp.int32(1)), buf1, sem1)

    def outer(ci, carry):
        acc = carry
        for b in range(2):
            c = 2 * ci + b
            h = wid + _NW * (c >> 2)
            pltpu.make_async_copy(chunk_src(c), bufs[b], sems[b]).wait()

            @pl.when((c & 3) == 0)
            def _zero():
                for r in range(_E, 2 * _E):
                    accs[r, :] = jnp.zeros((_E,), jnp.float32)

            acc = process_chunk(bufs[b], acc)
            nc = c + 2

            @pl.when(nc < _NCH)
            def _next():
                pltpu.async_copy(chunk_src(nc), bufs[b], sems[b])

            done = (c & 3) == 3

            @pl.when(done)
            def _flush():
                for j in range(_E):
                    accs[j, :] = acc[j]
                accs[2 * _E, :] = acc[_E]
                pltpu.sync_copy(accs, out_hbm.at[h])

            acc = [jnp.where(done, 0.0, a) for a in acc]
        return acc

    init = [jnp.zeros((_E,), jnp.float32)] * (_E + 1)
    lax.fori_loop(0, _NCH // 2, outer, init)


_sc_stats = functools.partial(
    pl.kernel,
    mesh=plsc.VectorSubcoreMesh(core_axis_name="c", subcore_axis_name="s"),
    out_type=jax.ShapeDtypeStruct((_HS, _ROWS, _E), jnp.float32),
    scratch_types=[
        pltpu.VMEM((_CTOK, _E), jnp.float32),
        pltpu.VMEM((_CTOK, _E), jnp.float32),
        pltpu.VMEM((_ROWS, _E), jnp.float32),
        pltpu.SemaphoreType.DMA,
        pltpu.SemaphoreType.DMA,
    ],
    compiler_params=pltpu.CompilerParams(
        needs_layout_passes=False, skip_device_barrier=True),
)(_sc_body)


def _ep_body(d_ref, o_ref):
    d = d_ref[...]
    hh = lax.broadcasted_iota(jnp.int32, (_HS, 1, 1), 0)
    gh = (hh // 2) % 4
    ps = d[:, 0:_E, :]
    cs = d[:, _E:2 * _E, :]
    zsum = jnp.sum(d[:, 2 * _E:2 * _E + 1, :])
    pg, cg = [], []
    for g in range(4):
        mk = (gh == g).astype(jnp.float32)
        pg.append(jnp.sum(jnp.sum(ps * mk, axis=0), axis=1))
        cg.append(jnp.sum(jnp.sum(cs * mk, axis=0), axis=1))
    prob = jnp.stack(pg)
    cnt = jnp.stack(cg)
    over = jnp.maximum(cnt - _CAP, 0.0)
    keep = jnp.minimum(cnt, _CAP)
    extra = jnp.sum(over, axis=1, keepdims=True)
    e0 = (lax.broadcasted_iota(jnp.int32, (4, _E), 1) == 0).astype(jnp.float32)
    cadj = keep + extra * e0
    aux = jnp.sum(cadj * prob) * (_E * _E / (4.0 * _E * _T * _T))
    z = zsum / (4.0 * _T)
    o_ref[...] = jnp.full((1, 1), 0.001 * (z + aux), jnp.float32)


def kernel(router_outputs, attention_mask):
    del attention_mask
    parts = _sc_stats(router_outputs.reshape(_HS * _TOK, _E))
    out = pl.pallas_call(
        _ep_body,
        out_shape=jax.ShapeDtypeStruct((1, 1), jnp.float32),
    )(parts)
    return out[0, 0]
